# baseline (speedup 1.0000x reference)
import functools

import jax
import jax.numpy as jnp
from jax import lax
from jax.experimental import pallas as pl

F32 = jnp.float32
_HI = lax.Precision.HIGHEST

_INTERPRET = False


def _rup(v, m):
    return (v + m - 1) // m * m


def _tapconv_body(*refs, taps, br, go, relu, n_in):
    grid_refs = refs[:n_in]
    w_ref = refs[n_in]
    b_ref = refs[n_in + 1]
    o_ref = refs[n_in + 2]
    base = pl.program_id(1) * br
    seen = []
    for t, (gi, off, og) in enumerate(taps):
        xs = grid_refs[gi][0, pl.ds(base + off, br), :]
        prod = lax.dot(xs, w_ref[t], preferred_element_type=F32)
        if og in seen:
            o_ref[0, og] += prod
        else:
            o_ref[0, og] = prod
            seen.append(og)
    b = b_ref[0]
    for g in range(go):
        r = o_ref[0, g] + b[None, :]
        if relu:
            r = jnp.maximum(r, 0.0)
        o_ref[0, g] = r


def _mm_body(x_ref, w_ref, b_ref, o_ref):
    prod = lax.dot(x_ref[0], w_ref[0], preferred_element_type=F32)
    o_ref[0, 0] = jnp.maximum(prod + b_ref[0][None, :], 0.0)


def _mm_conv(gridv, w_stack, bias, r_out, br_target=2048):
    n, _, cin = gridv.shape
    cout = w_stack.shape[2]
    nblk = max(1, -(-r_out // br_target))
    br = _rup(-(-r_out // nblk), 8)
    r_arr = nblk * br
    gridv = jnp.pad(gridv, ((0, 0), (0, r_arr - gridv.shape[1]), (0, 0)))
    out = pl.pallas_call(
        _mm_body,
        grid=(n, nblk),
        in_specs=[pl.BlockSpec((1, br, cin), lambda i, b: (i, b, 0)),
                  pl.BlockSpec(w_stack.shape, lambda i, b: (0, 0, 0)),
                  pl.BlockSpec((1, cout), lambda i, b: (0, 0))],
        out_specs=pl.BlockSpec((1, 1, br, cout), lambda i, b: (i, 0, b, 0)),
        out_shape=jax.ShapeDtypeStruct((n, 1, r_arr, cout), F32),
        interpret=_INTERPRET,
    )(gridv, w_stack, bias.reshape(1, -1))
    return out[:, :, :r_out, :]


def _tapconv(grids, w_stack, bias, taps, r_out, br_target=1024):
    n = grids[0].shape[0]
    _, cin, cout = w_stack.shape
    go = max(og for _, _, og in taps) + 1
    max_off = max(off for _, off, _ in taps)
    nblk = max(1, -(-r_out // br_target))
    br = _rup(-(-r_out // nblk), 8)
    r_arr = nblk * br
    r_need = _rup(r_arr + max_off, 8)
    grids = [jnp.pad(g, ((0, 0), (0, r_need - g.shape[1]), (0, 0)))
             if g.shape[1] < r_need else g[:, :r_need, :] for g in grids]
    body = functools.partial(_tapconv_body, taps=tuple(taps), br=br,
                             go=go, relu=True, n_in=len(grids))
    in_specs = [pl.BlockSpec((1, r_need, g.shape[2]), lambda i, b: (i, 0, 0))
                for g in grids]
    in_specs.append(pl.BlockSpec(w_stack.shape, lambda i, b: (0, 0, 0)))
    in_specs.append(pl.BlockSpec((1, cout), lambda i, b: (0, 0)))
    out = pl.pallas_call(
        body,
        grid=(n, nblk),
        in_specs=in_specs,
        out_specs=pl.BlockSpec((1, go, br, cout), lambda i, b: (i, 0, b, 0)),
        out_shape=jax.ShapeDtypeStruct((n, go, r_arr, cout), F32),
        interpret=_INTERPRET,
    )(*grids, w_stack, bias.reshape(1, -1))
    return out[:, :, :r_out, :]


def _conv_s1(x, w_oihw, bias, flip=False):
    n, h, w, c = x.shape
    hp, wp = h + 4, w + 4
    xp = jnp.pad(x, ((0, 0), (2, 2), (2, 2), (0, 0)))
    r_out = h * wp
    wts = []
    offs = []
    for dy in range(5):
        for dx in range(5):
            offs.append(dy * wp + dx)
            if flip:
                wts.append(w_oihw[:, :, 4 - dy, 4 - dx])
            else:
                wts.append(jnp.transpose(w_oihw[:, :, dy, dx]))
    w_stack = jnp.stack(wts)
    if c == 1:
        xf = xp.reshape(n, hp * wp)
        xf = jnp.pad(xf, ((0, 0), (0, r_out + max(offs) - hp * wp)))
        cols = jnp.stack([xf[:, o:o + r_out] for o in offs], axis=-1)
        w2 = w_stack[:, 0, :].reshape(1, 25, -1)
        out = _mm_conv(cols, w2, bias, r_out)
    else:
        gridv = xp.reshape(n, hp * wp, c)
        taps = [(0, o, 0) for o in offs]
        out = _tapconv([gridv], w_stack, bias, taps, r_out)
    return out[:, 0].reshape(n, h, wp, -1)[:, :, :w, :]


def _conv_s2(x, w_oihw, bias):
    n, h, w, c = x.shape
    hg, wg = (h + 1) // 2, (w + 1) // 2
    grids = []
    for py in range(2):
        for px in range(2):
            g = x[:, py::2, px::2, :]
            g = jnp.pad(g, ((0, 0), (0, hg - g.shape[1]),
                            (0, wg - g.shape[2]), (0, 0)))
            grids.append(g.reshape(n, hg * wg, c))
    ho, wo = (h - 5) // 2 + 1, (w - 5) // 2 + 1
    r_out = ho * wg
    taps = []
    wts = []
    for dy in range(5):
        for dx in range(5):
            taps.append(((dy % 2) * 2 + (dx % 2), (dy // 2) * wg + dx // 2, 0))
            wts.append(jnp.transpose(w_oihw[:, :, dy, dx]))
    w_stack = jnp.stack(wts)
    out = _tapconv(grids, w_stack, bias, taps, r_out)
    return out[:, 0].reshape(n, ho, wg, -1)[:, :, :wo, :]


def _convT_s2(x, w_iokk, bias):
    n, h, w, c = x.shape
    cout = w_iokk.shape[1]
    xp = jnp.pad(x, ((0, 0), (2, 2), (2, 2), (0, 0)))
    wg = w + 4
    hk, wk = h + 2, w + 2
    r_out = hk * wg
    taps = []
    wts = []
    for q in range(2):
        for r in range(2):
            for my in range(3 - q):
                for mx in range(3 - r):
                    taps.append((0, (2 - my) * wg + (2 - mx), q * 2 + r))
                    wts.append(w_iokk[:, :, 2 * my + q, 2 * mx + r])
    w_stack = jnp.stack(wts)
    gridv = xp.reshape(n, (h + 4) * wg, c)
    br_target = 512 if cout == 1 else 1024
    out = _tapconv([gridv], w_stack, bias, taps, r_out, br_target=br_target)
    out = out.reshape(n, 2, 2, hk, wg, cout)[:, :, :, :, :wk, :]
    out = jnp.transpose(out, (0, 3, 1, 4, 2, 5)).reshape(
        n, 2 * hk, 2 * wk, cout)
    return out[:, :2 * h + 3, :2 * w + 3, :]


def _stencil_body(x_ref, w_ref, b_ref, o_ref, *, h_out, w_out):
    acc = None
    for dy in range(5):
        for dx in range(5):
            xs = x_ref[0, dy:dy + h_out, dx:dx + w_out]
            xs = xs.astype(jnp.bfloat16).astype(F32)
            wv = w_ref[dy, dx].astype(jnp.bfloat16).astype(F32)
            p = wv * xs
            acc = p if acc is None else acc + p
    o_ref[0] = jnp.maximum(acc + b_ref[0, 0], 0.0)


def _conv_s1_1ch(x, w_iokk, bias):
    n, h, w, _ = x.shape
    xp = jnp.pad(x[..., 0], ((0, 0), (2, 2), (2, 2 + 8)))
    w_out = w + 8
    wf = w_iokk[0, 0, ::-1, ::-1]
    body = functools.partial(_stencil_body, h_out=h, w_out=w_out)
    out = pl.pallas_call(
        body,
        grid=(n,),
        in_specs=[pl.BlockSpec((1, h + 4, w + 4 + 8), lambda i: (i, 0, 0)),
                  pl.BlockSpec((5, 5), lambda i: (0, 0)),
                  pl.BlockSpec((1, 1), lambda i: (0, 0))],
        out_specs=pl.BlockSpec((1, h, w_out), lambda i: (i, 0, 0)),
        out_shape=jax.ShapeDtypeStruct((n, h, w_out), F32),
        interpret=_INTERPRET,
    )(xp, wf, bias.reshape(1, 1))
    return out[:, :, :w, None]


def _vq_body(x_ref, e_ref, idx_ref, ab_ref, *, rows, ncodes):
    x = x_ref[0]
    e = e_ref[...]
    xsq = jnp.sum(x * x, axis=1, keepdims=True)
    esq = jnp.sum(e * e, axis=0, keepdims=True)
    xe = lax.dot(x, e, precision=_HI, preferred_element_type=F32)
    d2 = xsq - 2.0 * xe + esq
    iota = lax.broadcasted_iota(jnp.int32, (rows, ncodes), 1)
    mind = jnp.min(d2, axis=1, keepdims=True)
    idx = jnp.min(jnp.where(d2 == mind, iota, ncodes), axis=1)
    idx_ref[0, 0] = idx
    oh = (iota == idx[:, None]).astype(F32)
    ab = lax.dot_general(oh, e, (((1,), (1,)), ((), ())),
                         precision=_HI, preferred_element_type=F32)
    ab_ref[0] = ab


def _vq(bb_rows, e):
    n, rows, d = bb_rows.shape
    ncodes = e.shape[1]
    body = functools.partial(_vq_body, rows=rows, ncodes=ncodes)
    idx, ab = pl.pallas_call(
        body,
        grid=(n,),
        in_specs=[pl.BlockSpec((1, rows, d), lambda i: (i, 0, 0)),
                  pl.BlockSpec(e.shape, lambda i: (0, 0))],
        out_specs=[pl.BlockSpec((1, 1, rows), lambda i: (i, 0, 0)),
                   pl.BlockSpec((1, rows, d), lambda i: (i, 0, 0))],
        out_shape=[jax.ShapeDtypeStruct((n, 1, rows), jnp.int32),
                   jax.ShapeDtypeStruct((n, rows, d), F32)],
        interpret=_INTERPRET,
    )(bb_rows, e)
    return idx[:, 0], ab


def kernel(x, w1a, b1a, w1b, b1b, w2a, b2a, w2b, b2b, w3a, b3a, w3b, b3b,
           wt3a, bt3a, wt3b, bt3b, wt2a, bt2a, wt2b, bt2b, wt1a, bt1a,
           wt1b, bt1b, e):
    n = x.shape[0]
    h = jnp.transpose(x, (0, 2, 3, 1))
    h = _conv_s1(h, w1a, b1a)
    h = _conv_s2(h, w1b, b1b)
    h = _conv_s1(h, w2a, b2a)
    h = _conv_s2(h, w2b, b2b)
    h = _conv_s1(h, w3a, b3a)
    bb = _conv_s2(h, w3b, b3b)
    hw = bb.shape[1] * bb.shape[2]
    idx, ab_rows = _vq(bb.reshape(n, hw, 64), e)
    idxs = idx.reshape(n, bb.shape[1], bb.shape[2])
    ab = ab_rows.reshape(n, bb.shape[1], bb.shape[2], 64)
    d = _convT_s2(ab, wt3a, bt3a)
    d = _conv_s1(d, wt3b, bt3b, flip=True)
    d = _convT_s2(d, wt2a, bt2a)
    d = _conv_s1(d, wt2b, bt2b, flip=True)
    d = _convT_s2(d, wt1a, bt1a)
    d = _conv_s1_1ch(d, wt1b, bt1b)
    recon = jnp.transpose(d, (0, 3, 1, 2))
    bb_nchw = jnp.transpose(bb, (0, 3, 1, 2))
    ab_nchw = jnp.transpose(ab, (0, 3, 1, 2))
    return (recon, idxs, bb_nchw, ab_nchw)
